# Initial kernel scaffold; baseline (speedup 1.0000x reference)
import jax
import jax.numpy as jnp
from jax.experimental import pallas as pl


def kernel(batch, table):
    raise NotImplementedError("write your pallas kernel here")


# speedup vs baseline: 2.7460x; 2.7460x over previous
import functools

import jax
import jax.numpy as jnp
from jax import lax
from jax.experimental import pallas as pl
from jax.experimental.pallas import tpu as pltpu
from jax.experimental.pallas import tpu_sc as plsc

B = 16384
L = 50
D = 32
NW = 32
EPW = B // NW
CB = 32
NCH = EPW // CB
ROWS = CB * L
SUBI = 100
NSUB = ROWS // SUBI
HALF = D // 2


def kernel(batch, table):
    idx2d = batch.reshape(B * L // SUBI, SUBI)
    mesh = plsc.VectorSubcoreMesh(core_axis_name="c", subcore_axis_name="s")

    @functools.partial(
        pl.kernel,
        mesh=mesh,
        out_type=jax.ShapeDtypeStruct((B, D), jnp.float32),
        scratch_types=[
            pltpu.VMEM((NSUB, SUBI), jnp.int32),
            pltpu.VMEM((ROWS, D), jnp.float32),
            pltpu.VMEM((CB, D), jnp.float32),
            pltpu.SemaphoreType.DMA,
        ],
        compiler_params=pltpu.CompilerParams(use_tc_tiling_on_sc=False),
    )
    def bow(idx_hbm, table_hbm, out_hbm, idx_v, rows_v, out_v, sem):
        wid = lax.axis_index("s") * 2 + lax.axis_index("c")

        def chunk_body(c, carry):
            irow0 = wid * (NCH * NSUB) + c * NSUB
            pltpu.sync_copy(idx_hbm.at[pl.ds(irow0, NSUB)], idx_v)
            copies = [
                pltpu.async_copy(
                    table_hbm.at[idx_v.at[j]],
                    rows_v.at[pl.ds(j * SUBI, SUBI)],
                    sem,
                )
                for j in range(NSUB)
            ]
            for cp in copies:
                cp.wait()

            def elem_body(e, carry2):
                r0 = e * L
                a0 = rows_v[r0, 0:HALF] + rows_v[r0 + 1, 0:HALF]
                b0 = rows_v[r0, HALF:D] + rows_v[r0 + 1, HALF:D]
                a1 = rows_v[r0 + 2, 0:HALF] + rows_v[r0 + 3, 0:HALF]
                b1 = rows_v[r0 + 2, HALF:D] + rows_v[r0 + 3, HALF:D]
                for l in range(4, L, 2):
                    a0 = a0 + rows_v[r0 + l, 0:HALF]
                    b0 = b0 + rows_v[r0 + l, HALF:D]
                    a1 = a1 + rows_v[r0 + l + 1, 0:HALF]
                    b1 = b1 + rows_v[r0 + l + 1, HALF:D]
                out_v[e, 0:HALF] = (a0 + a1) * (1.0 / L)
                out_v[e, HALF:D] = (b0 + b1) * (1.0 / L)
                return carry2

            lax.fori_loop(0, CB, elem_body, 0)
            obase = wid * EPW + c * CB
            pltpu.sync_copy(out_v, out_hbm.at[pl.ds(obase, CB)])
            return carry

        lax.fori_loop(0, NCH, chunk_body, 0)

    return bow(idx2d, table)


# speedup vs baseline: 2.7510x; 1.0018x over previous
import functools

import jax
import jax.numpy as jnp
from jax import lax
from jax.experimental import pallas as pl
from jax.experimental.pallas import tpu as pltpu
from jax.experimental.pallas import tpu_sc as plsc

B = 16384
L = 50
D = 32
NW = 32
EPW = B // NW
CB = 32
NCH = EPW // CB
ROWS = CB * L
SUBI = 80
NSUB = ROWS // SUBI
HALF = D // 2


def kernel(batch, table):
    idx_flat = batch.reshape(B * L)
    mesh = plsc.VectorSubcoreMesh(core_axis_name="c", subcore_axis_name="s")

    @functools.partial(
        pl.kernel,
        mesh=mesh,
        out_type=jax.ShapeDtypeStruct((B * D,), jnp.float32),
        scratch_types=[
            pltpu.VMEM((ROWS,), jnp.int32),
            pltpu.VMEM((ROWS, D), jnp.float32),
            pltpu.VMEM((CB * D,), jnp.float32),
            pltpu.SemaphoreType.DMA,
        ],
        compiler_params=pltpu.CompilerParams(use_tc_tiling_on_sc=False),
    )
    def bow(idx_hbm, table_hbm, out_hbm, idx_v, rows_v, out_v, sem):
        wid = lax.axis_index("s") * 2 + lax.axis_index("c")

        def chunk_body(c, carry):
            i0 = (wid * EPW + c * CB) * L
            pltpu.sync_copy(idx_hbm.at[pl.ds(i0, ROWS)], idx_v)
            copies = [
                pltpu.async_copy(
                    table_hbm.at[idx_v.at[pl.ds(j * SUBI, SUBI)]],
                    rows_v.at[pl.ds(j * SUBI, SUBI)],
                    sem,
                )
                for j in range(NSUB)
            ]
            for cp in copies:
                cp.wait()

            def elem_body(e, carry2):
                r0 = e * L
                a0 = rows_v[r0, 0:HALF] + rows_v[r0 + 1, 0:HALF]
                b0 = rows_v[r0, HALF:D] + rows_v[r0 + 1, HALF:D]
                a1 = rows_v[r0 + 2, 0:HALF] + rows_v[r0 + 3, 0:HALF]
                b1 = rows_v[r0 + 2, HALF:D] + rows_v[r0 + 3, HALF:D]
                for l in range(4, L, 2):
                    a0 = a0 + rows_v[r0 + l, 0:HALF]
                    b0 = b0 + rows_v[r0 + l, HALF:D]
                    a1 = a1 + rows_v[r0 + l + 1, 0:HALF]
                    b1 = b1 + rows_v[r0 + l + 1, HALF:D]
                o0 = e * D
                out_v[pl.ds(o0, HALF)] = (a0 + a1) * (1.0 / L)
                out_v[pl.ds(o0 + HALF, HALF)] = (b0 + b1) * (1.0 / L)
                return carry2

            lax.fori_loop(0, CB, elem_body, 0)
            obase = (wid * EPW + c * CB) * D
            pltpu.sync_copy(out_v, out_hbm.at[pl.ds(obase, CB * D)])
            return carry

        lax.fori_loop(0, NCH, chunk_body, 0)

    return bow(idx_flat, table).reshape(B, D)


# speedup vs baseline: 2.7731x; 1.0080x over previous
import functools

import jax
import jax.numpy as jnp
from jax import lax
from jax.experimental import pallas as pl
from jax.experimental.pallas import tpu as pltpu
from jax.experimental.pallas import tpu_sc as plsc

B = 16384
L = 50
D = 32
NW = 32
EPW = B // NW
CB = 32
NCH = EPW // CB
ROWS = CB * L
HALF = D // 2


def kernel(batch, table):
    idx_bag_major = batch.T.reshape(B * L)
    mesh = plsc.VectorSubcoreMesh(core_axis_name="c", subcore_axis_name="s")

    @functools.partial(
        pl.kernel,
        mesh=mesh,
        out_type=jax.ShapeDtypeStruct((B * D,), jnp.float32),
        scratch_types=[
            pltpu.VMEM((ROWS,), jnp.int32),
            pltpu.VMEM((ROWS, D), jnp.float32),
            pltpu.VMEM((CB * D,), jnp.float32),
            pltpu.SemaphoreType.DMA,
            pltpu.SemaphoreType.DMA,
        ],
        compiler_params=pltpu.CompilerParams(use_tc_tiling_on_sc=False),
    )
    def bow(idx_hbm, table_hbm, out_hbm, idx_v, rows_v, out_v, isem, gsem):
        wid = lax.axis_index("s") * 2 + lax.axis_index("c")

        def chunk_body(c, carry):
            b0 = wid * EPW + c * CB
            icopies = [
                pltpu.async_copy(
                    idx_hbm.at[pl.ds(l * B + b0, CB)],
                    idx_v.at[pl.ds(l * CB, CB)],
                    isem,
                )
                for l in range(L)
            ]
            for cp in icopies:
                cp.wait()
            gcopies = [
                pltpu.async_copy(
                    table_hbm.at[idx_v.at[pl.ds(l * CB, CB)]],
                    rows_v.at[pl.ds(l * CB, CB)],
                    gsem,
                )
                for l in range(L)
            ]
            for cp in gcopies:
                cp.wait()

            def elem_body(e, carry2):
                a0 = rows_v[e, 0:HALF] + rows_v[e + CB, 0:HALF]
                b0_ = rows_v[e, HALF:D] + rows_v[e + CB, HALF:D]
                a1 = rows_v[e + 2 * CB, 0:HALF] + rows_v[e + 3 * CB, 0:HALF]
                b1 = rows_v[e + 2 * CB, HALF:D] + rows_v[e + 3 * CB, HALF:D]
                for l in range(4, L, 2):
                    a0 = a0 + rows_v[e + l * CB, 0:HALF]
                    b0_ = b0_ + rows_v[e + l * CB, HALF:D]
                    a1 = a1 + rows_v[e + (l + 1) * CB, 0:HALF]
                    b1 = b1 + rows_v[e + (l + 1) * CB, HALF:D]
                o0 = e * D
                out_v[pl.ds(o0, HALF)] = (a0 + a1) * (1.0 / L)
                out_v[pl.ds(o0 + HALF, HALF)] = (b0_ + b1) * (1.0 / L)
                return carry2

            lax.fori_loop(0, CB, elem_body, 0)
            pltpu.sync_copy(out_v, out_hbm.at[pl.ds(b0 * D, CB * D)])
            return carry

        lax.fori_loop(0, NCH, chunk_body, 0)

    return bow(idx_bag_major, table).reshape(B, D)
